# baseline (speedup 1.0000x reference)
import functools

import jax
import jax.numpy as jnp
from jax import lax
from jax.experimental import pallas as pl
from jax.experimental.pallas import tpu as pltpu
from jax.experimental.pallas import tpu_sc as plsc

NC = 2
NS = 16
NW = NC * NS

BATCH = 16384
FIELDS = 26
DIM = 32
B = BATCH * FIELDS
BPW = B // NW
CHUNK = 128
CPW = BPW // CHUNK
NBUF = 8
NGRP = CPW // NBUF

NB1 = 4

assert CPW * CHUNK == BPW and NGRP * NBUF == CPW


def _t_body(tt_hbm, st_hbm, inb, outb, gsem, osem):
    c = lax.axis_index("c")
    s = lax.axis_index("s")
    wid = s * NC + c
    nblk = jnp.where(wid < 5, 245, 244)
    rt0 = 244 * wid + jnp.minimum(wid, 5)

    iot = lax.iota(jnp.int32, 16)
    pats = [((iot + v * 16) % 32) * 128 + (iot + v * 16) // 32
            for v in range(8)]
    prow = [p // 128 for p in pats]
    pcol = [lax.rem(p, 128) for p in pats]

    def in_start(b, rt):
        pltpu.make_async_copy(
            tt_hbm.at[:, pl.ds(rt * 128, 128)], inb.at[b], gsem.at[b]).start()

    def in_wait(b, rt):
        pltpu.make_async_copy(
            tt_hbm.at[:, pl.ds(rt * 128, 128)], inb.at[b], gsem.at[b]).wait()

    def out_start(b, rt):
        pltpu.make_async_copy(
            outb.at[b], st_hbm.at[pl.ds(rt * 32, 32)], osem.at[b]).start()

    def out_wait(b, rt):
        pltpu.make_async_copy(
            outb.at[b], st_hbm.at[pl.ds(rt * 32, 32)], osem.at[b]).wait()

    def transpose(b):
        @plsc.parallel_loop(0, 32, unroll=8)
        def _r4(r4):
            r4vec = jnp.full((16,), r4, jnp.int32)
            vals = [plsc.load_gather(inb.at[b], [prow[v], pcol[v] + r4 * 4])
                    for v in range(8)]
            for v in range(8):
                plsc.store_scatter(outb.at[b], [r4vec, iot + v * 16], vals[v])

    for b in range(NB1):
        @pl.when(b < nblk)
        def _():
            in_start(b, rt0 + b)

    @pl.loop(0, 248, step=NB1)
    def _grp(g):
        for b in range(NB1):
            k = g + b

            @pl.when(k < nblk)
            def _():
                in_wait(b, rt0 + k)
                transpose(b)
                out_start(b, rt0 + k)
        for b in range(NB1):
            k = g + b

            @pl.when(k < nblk)
            def _():
                out_wait(b, rt0 + k)

            @pl.when(k + NB1 < nblk)
            def _():
                in_start(b, rt0 + k + NB1)


def _g_body(x_hbm, st_hbm, out_hbm, idx_v, gbuf, gsem, osem):
    c = lax.axis_index("c")
    s = lax.axis_index("s")
    wid = s * NC + c
    base = wid * BPW

    pltpu.sync_copy(x_hbm.at[pl.ds(wid * CPW, CPW)], idx_v)

    def g_start(b, j):
        pltpu.make_async_copy(
            st_hbm.at[idx_v.at[j]], gbuf.at[b], gsem.at[b]).start()

    def g_wait(b, j):
        pltpu.make_async_copy(
            st_hbm.at[idx_v.at[j]], gbuf.at[b], gsem.at[b]).wait()

    def o_start(b, j):
        pltpu.make_async_copy(
            gbuf.at[b], out_hbm.at[pl.ds(base + j * CHUNK, CHUNK)],
            osem.at[b]).start()

    def o_wait(b, j):
        pltpu.make_async_copy(
            gbuf.at[b], out_hbm.at[pl.ds(base + j * CHUNK, CHUNK)],
            osem.at[b]).wait()

    for b in range(NBUF):
        g_start(b, b)

    @pl.loop(0, (NGRP - 1) * NBUF, step=NBUF)
    def _grp(g):
        for b in range(NBUF):
            g_wait(b, g + b)
            o_start(b, g + b)
        for b in range(NBUF):
            o_wait(b, g + b)
            g_start(b, g + NBUF + b)

    gl = (NGRP - 1) * NBUF
    for b in range(NBUF):
        g_wait(b, gl + b)
        o_start(b, gl + b)
    for b in range(NBUF):
        o_wait(b, gl + b)


def _mesh():
    return plsc.VectorSubcoreMesh(
        core_axis_name="c", subcore_axis_name="s",
        num_cores=NC, num_subcores=NS)


@jax.jit
def _lookup(x_flat, tt):
    k1 = functools.partial(
        pl.kernel,
        out_type=jax.ShapeDtypeStruct((250016, 128), jnp.float32),
        mesh=_mesh(),
        compiler_params=pltpu.CompilerParams(needs_layout_passes=False),
        scratch_types=[
            pltpu.VMEM((NB1, 32, 128), jnp.float32),
            pltpu.VMEM((NB1, 32, 128), jnp.float32),
            pltpu.SemaphoreType.DMA((NB1,)),
            pltpu.SemaphoreType.DMA((NB1,)),
        ],
    )(_t_body)
    staging = k1(tt).reshape(1000064, 32)
    k2 = functools.partial(
        pl.kernel,
        out_type=jax.ShapeDtypeStruct((B, DIM), jnp.float32),
        mesh=_mesh(),
        compiler_params=pltpu.CompilerParams(use_tc_tiling_on_sc=False),
        scratch_types=[
            pltpu.VMEM((CPW, CHUNK), jnp.int32),
            pltpu.VMEM((NBUF, CHUNK, DIM), jnp.float32),
            pltpu.SemaphoreType.DMA((NBUF,)),
            pltpu.SemaphoreType.DMA((NBUF,)),
        ],
    )(_g_body)
    return k2(x_flat, staging)


def kernel(x, table):
    x_flat = x.reshape(NW * CPW, CHUNK).astype(jnp.int32)
    out = _lookup(x_flat, table.T)
    return out.reshape(BATCH, FIELDS, DIM)


# speedup vs baseline: 1.0961x; 1.0961x over previous
import functools

import jax
import jax.numpy as jnp
from jax import lax
from jax.experimental import pallas as pl
from jax.experimental.pallas import tpu as pltpu
from jax.experimental.pallas import tpu_sc as plsc

NC = 2
NS = 16
NW = NC * NS

BATCH = 16384
FIELDS = 26
DIM = 32
B = BATCH * FIELDS
BPW = B // NW
CHUNK = 128
CPW = BPW // CHUNK
NBUF = 8
NGRP = CPW // NBUF

NB1 = 4

assert CPW * CHUNK == BPW and NGRP * NBUF == CPW


def _t_body(tt_hbm, st_hbm, inb, outb, gsem, osem):
    c = lax.axis_index("c")
    s = lax.axis_index("s")
    wid = s * NC + c
    nblk = jnp.where(wid < 5, 245, 244)
    rt0 = 244 * wid + jnp.minimum(wid, 5)

    iot = lax.iota(jnp.int32, 16)
    lbase = [((v % 2) * 16 + iot) * 128 + v // 2 for v in range(8)]
    srow = [jnp.full((16,), 0, jnp.int32) for _ in range(1)]
    scol = [iot + v * 16 for v in range(8)]

    def in_start(b, rt):
        pltpu.make_async_copy(
            tt_hbm.at[:, pl.ds(rt * 128, 128)], inb.at[b], gsem.at[b]).start()

    def in_wait(b, rt):
        pltpu.make_async_copy(
            tt_hbm.at[:, pl.ds(rt * 128, 128)], inb.at[b], gsem.at[b]).wait()

    def out_start(b, rt):
        pltpu.make_async_copy(
            outb.at[b], st_hbm.at[pl.ds(rt * 32, 32)], osem.at[b]).start()

    def out_wait(b, rt):
        pltpu.make_async_copy(
            outb.at[b], st_hbm.at[pl.ds(rt * 32, 32)], osem.at[b]).wait()

    def transpose(b):
        flat_in = [lb // 128 for lb in lbase]
        flat_ic = [lax.rem(lb, 128) for lb in lbase]

        @plsc.parallel_loop(0, 32, unroll=4)
        def _r4(r4):
            r4vec = jnp.full((16,), r4, jnp.int32)
            vals = [plsc.load_gather(inb.at[b], [flat_in[v], flat_ic[v] + r4 * 4])
                    for v in range(8)]
            for v in range(8):
                plsc.store_scatter(outb.at[b], [r4vec, scol[v]], vals[v])

    for b in range(NB1):
        @pl.when(b < nblk)
        def _():
            in_start(b, rt0 + b)

    @pl.loop(0, 248, step=NB1)
    def _grp(g):
        for b in range(NB1):
            k = g + b

            @pl.when(k < nblk)
            def _():
                in_wait(b, rt0 + k)
                transpose(b)
                out_start(b, rt0 + k)
        for b in range(NB1):
            k = g + b

            @pl.when(k < nblk)
            def _():
                out_wait(b, rt0 + k)

            @pl.when(k + NB1 < nblk)
            def _():
                in_start(b, rt0 + k + NB1)


def _g_body(x_hbm, st_hbm, out_hbm, idx_v, gbuf, gsem, osem):
    c = lax.axis_index("c")
    s = lax.axis_index("s")
    wid = s * NC + c
    base = wid * BPW

    pltpu.sync_copy(x_hbm.at[pl.ds(wid * CPW, CPW)], idx_v)

    def g_start(b, j):
        pltpu.make_async_copy(
            st_hbm.at[idx_v.at[j]], gbuf.at[b], gsem.at[b]).start()

    def g_wait(b, j):
        pltpu.make_async_copy(
            st_hbm.at[idx_v.at[j]], gbuf.at[b], gsem.at[b]).wait()

    def o_start(b, j):
        pltpu.make_async_copy(
            gbuf.at[b], out_hbm.at[pl.ds(base + j * CHUNK, CHUNK)],
            osem.at[b]).start()

    def o_wait(b, j):
        pltpu.make_async_copy(
            gbuf.at[b], out_hbm.at[pl.ds(base + j * CHUNK, CHUNK)],
            osem.at[b]).wait()

    for b in range(NBUF):
        g_start(b, b)

    @pl.loop(0, (NGRP - 1) * NBUF, step=NBUF)
    def _grp(g):
        for b in range(NBUF):
            g_wait(b, g + b)
            o_start(b, g + b)
        for b in range(NBUF):
            o_wait(b, g + b)
            g_start(b, g + NBUF + b)

    gl = (NGRP - 1) * NBUF
    for b in range(NBUF):
        g_wait(b, gl + b)
        o_start(b, gl + b)
    for b in range(NBUF):
        o_wait(b, gl + b)


def _mesh():
    return plsc.VectorSubcoreMesh(
        core_axis_name="c", subcore_axis_name="s",
        num_cores=NC, num_subcores=NS)


@jax.jit
def _lookup(x_flat, tt):
    k1 = functools.partial(
        pl.kernel,
        out_type=jax.ShapeDtypeStruct((250016, 128), jnp.float32),
        mesh=_mesh(),
        compiler_params=pltpu.CompilerParams(needs_layout_passes=False),
        scratch_types=[
            pltpu.VMEM((NB1, 32, 128), jnp.float32),
            pltpu.VMEM((NB1, 32, 128), jnp.float32),
            pltpu.SemaphoreType.DMA((NB1,)),
            pltpu.SemaphoreType.DMA((NB1,)),
        ],
    )(_t_body)
    staging = k1(tt).reshape(1000064, 32)
    k2 = functools.partial(
        pl.kernel,
        out_type=jax.ShapeDtypeStruct((B, DIM), jnp.float32),
        mesh=_mesh(),
        compiler_params=pltpu.CompilerParams(use_tc_tiling_on_sc=False),
        scratch_types=[
            pltpu.VMEM((CPW, CHUNK), jnp.int32),
            pltpu.VMEM((NBUF, CHUNK, DIM), jnp.float32),
            pltpu.SemaphoreType.DMA((NBUF,)),
            pltpu.SemaphoreType.DMA((NBUF,)),
        ],
    )(_g_body)
    return k2(x_flat, staging)


def kernel(x, table):
    x_flat = x.reshape(NW * CPW, CHUNK).astype(jnp.int32)
    out = _lookup(x_flat, table.T)
    return out.reshape(BATCH, FIELDS, DIM)


# speedup vs baseline: 1.3338x; 1.2168x over previous
import functools

import jax
import jax.numpy as jnp
from jax import lax
from jax.experimental import pallas as pl
from jax.experimental.pallas import tpu as pltpu
from jax.experimental.pallas import tpu_sc as plsc

NC = 2
NS = 16
NW = NC * NS

BATCH = 16384
FIELDS = 26
DIM = 32
B = BATCH * FIELDS
BPW = B // NW
CHUNK = 128
CPW = BPW // CHUNK
NBUF = 8
NGRP = CPW // NBUF

NB1 = 4

assert CPW * CHUNK == BPW and NGRP * NBUF == CPW


def _t_body(tt_hbm, st_hbm, inb, outb, gsem, osem):
    c = lax.axis_index("c")
    s = lax.axis_index("s")
    wid = s * NC + c
    nblk = jnp.where(wid < 5, 245, 244)
    rt0 = 244 * wid + jnp.minimum(wid, 5)

    iot = lax.iota(jnp.int32, 16)
    ldrow = [16 * h + iot for h in range(2)]
    ricb = [lax.rem(iot + t, 16) for t in range(16)]
    strow = [r // 4 for r in ricb]
    stcol = [[lax.rem(ricb[t], 4) * 32 + 16 * h + iot
              for t in range(16)] for h in range(2)]

    def in_start(b, rt):
        pltpu.make_async_copy(
            tt_hbm.at[:, pl.ds(rt * 128, 128)], inb.at[b], gsem.at[b]).start()

    def in_wait(b, rt):
        pltpu.make_async_copy(
            tt_hbm.at[:, pl.ds(rt * 128, 128)], inb.at[b], gsem.at[b]).wait()

    def out_start(b, rt):
        pltpu.make_async_copy(
            outb.at[b], st_hbm.at[pl.ds(rt * 32, 32)], osem.at[b]).start()

    def out_wait(b, rt):
        pltpu.make_async_copy(
            outb.at[b], st_hbm.at[pl.ds(rt * 32, 32)], osem.at[b]).wait()

    def transpose(b):
        @plsc.parallel_loop(0, 8, unroll=2)
        def _g(g):
            ric = [ricb[t] + 16 * g for t in range(16)]
            srw = [strow[t] + 4 * g for t in range(16)]
            for h in range(2):
                vals = [plsc.load_gather(inb.at[b], [ldrow[h], ric[t]])
                        for t in range(16)]
                for t in range(16):
                    plsc.store_scatter(outb.at[b], [srw[t], stcol[h][t]], vals[t])

    for b in range(NB1):
        @pl.when(b < nblk)
        def _():
            in_start(b, rt0 + b)

    @pl.loop(0, 248, step=NB1)
    def _grp(g):
        for b in range(NB1):
            k = g + b

            @pl.when(k < nblk)
            def _():
                in_wait(b, rt0 + k)
                transpose(b)
                out_start(b, rt0 + k)
        for b in range(NB1):
            k = g + b

            @pl.when(k < nblk)
            def _():
                out_wait(b, rt0 + k)

            @pl.when(k + NB1 < nblk)
            def _():
                in_start(b, rt0 + k + NB1)


def _g_body(x_hbm, st_hbm, out_hbm, idx_v, gbuf, gsem, osem):
    c = lax.axis_index("c")
    s = lax.axis_index("s")
    wid = s * NC + c
    base = wid * BPW

    pltpu.sync_copy(x_hbm.at[pl.ds(wid * CPW, CPW)], idx_v)

    def g_start(b, j):
        pltpu.make_async_copy(
            st_hbm.at[idx_v.at[j]], gbuf.at[b], gsem.at[b]).start()

    def g_wait(b, j):
        pltpu.make_async_copy(
            st_hbm.at[idx_v.at[j]], gbuf.at[b], gsem.at[b]).wait()

    def o_start(b, j):
        pltpu.make_async_copy(
            gbuf.at[b], out_hbm.at[pl.ds(base + j * CHUNK, CHUNK)],
            osem.at[b]).start()

    def o_wait(b, j):
        pltpu.make_async_copy(
            gbuf.at[b], out_hbm.at[pl.ds(base + j * CHUNK, CHUNK)],
            osem.at[b]).wait()

    for b in range(NBUF):
        g_start(b, b)

    @pl.loop(0, (NGRP - 1) * NBUF, step=NBUF)
    def _grp(g):
        for b in range(NBUF):
            g_wait(b, g + b)
            o_start(b, g + b)
        for b in range(NBUF):
            o_wait(b, g + b)
            g_start(b, g + NBUF + b)

    gl = (NGRP - 1) * NBUF
    for b in range(NBUF):
        g_wait(b, gl + b)
        o_start(b, gl + b)
    for b in range(NBUF):
        o_wait(b, gl + b)


def _mesh():
    return plsc.VectorSubcoreMesh(
        core_axis_name="c", subcore_axis_name="s",
        num_cores=NC, num_subcores=NS)


@jax.jit
def _lookup(x_flat, tt):
    k1 = functools.partial(
        pl.kernel,
        out_type=jax.ShapeDtypeStruct((250016, 128), jnp.float32),
        mesh=_mesh(),
        compiler_params=pltpu.CompilerParams(needs_layout_passes=False),
        scratch_types=[
            pltpu.VMEM((NB1, 32, 128), jnp.float32),
            pltpu.VMEM((NB1, 32, 128), jnp.float32),
            pltpu.SemaphoreType.DMA((NB1,)),
            pltpu.SemaphoreType.DMA((NB1,)),
        ],
    )(_t_body)
    staging = k1(tt).reshape(1000064, 32)
    k2 = functools.partial(
        pl.kernel,
        out_type=jax.ShapeDtypeStruct((B, DIM), jnp.float32),
        mesh=_mesh(),
        compiler_params=pltpu.CompilerParams(use_tc_tiling_on_sc=False),
        scratch_types=[
            pltpu.VMEM((CPW, CHUNK), jnp.int32),
            pltpu.VMEM((NBUF, CHUNK, DIM), jnp.float32),
            pltpu.SemaphoreType.DMA((NBUF,)),
            pltpu.SemaphoreType.DMA((NBUF,)),
        ],
    )(_g_body)
    return k2(x_flat, staging)


def kernel(x, table):
    x_flat = x.reshape(NW * CPW, CHUNK).astype(jnp.int32)
    out = _lookup(x_flat, table.T)
    return out.reshape(BATCH, FIELDS, DIM)


# speedup vs baseline: 1.6139x; 1.2100x over previous
import functools

import jax
import jax.numpy as jnp
from jax import lax
from jax.experimental import pallas as pl
from jax.experimental.pallas import tpu as pltpu
from jax.experimental.pallas import tpu_sc as plsc

NC = 2
NS = 16
NW = NC * NS

BATCH = 16384
FIELDS = 26
DIM = 32
B = BATCH * FIELDS
BPW = B // NW
CHUNK = 128
CPW = BPW // CHUNK
NBUF = 4
NGRP = CPW // NBUF

NB1 = 4

assert CPW * CHUNK == BPW and NGRP * NBUF == CPW


def _t_body(tt_hbm, st_hbm, inb, outb, gsem, osem):
    c = lax.axis_index("c")
    s = lax.axis_index("s")
    wid = s * NC + c
    nblk = jnp.where(wid < 5, 245, 244)
    rt0 = 244 * wid + jnp.minimum(wid, 5)

    iot = lax.iota(jnp.int32, 16)
    ldrow = [16 * h + iot for h in range(2)]
    ricb = [lax.rem(iot + t, 16) for t in range(16)]
    strow = [r // 4 for r in ricb]
    stcol = [[lax.rem(ricb[t], 4) * 32 + 16 * h + iot
              for t in range(16)] for h in range(2)]

    def in_start(b, rt):
        pltpu.make_async_copy(
            tt_hbm.at[:, pl.ds(rt * 128, 128)], inb.at[b], gsem.at[b]).start()

    def in_wait(b, rt):
        pltpu.make_async_copy(
            tt_hbm.at[:, pl.ds(rt * 128, 128)], inb.at[b], gsem.at[b]).wait()

    def out_start(b, rt):
        pltpu.make_async_copy(
            outb.at[b], st_hbm.at[pl.ds(rt * 32, 32)], osem.at[b]).start()

    def out_wait(b, rt):
        pltpu.make_async_copy(
            outb.at[b], st_hbm.at[pl.ds(rt * 32, 32)], osem.at[b]).wait()

    def transpose(b):
        @plsc.parallel_loop(0, 8, unroll=2)
        def _g(g):
            ric = [ricb[t] + 16 * g for t in range(16)]
            srw = [strow[t] + 4 * g for t in range(16)]
            for h in range(2):
                vals = [plsc.load_gather(inb.at[b], [ldrow[h], ric[t]])
                        for t in range(16)]
                for t in range(16):
                    plsc.store_scatter(outb.at[b], [srw[t], stcol[h][t]], vals[t])

    for b in range(NB1):
        @pl.when(b < nblk)
        def _():
            in_start(b, rt0 + b)

    @pl.loop(0, 248, step=NB1)
    def _grp(g):
        for b in range(NB1):
            k = g + b

            @pl.when(k < nblk)
            def _():
                in_wait(b, rt0 + k)
                transpose(b)
                out_start(b, rt0 + k)
        for b in range(NB1):
            k = g + b

            @pl.when(k < nblk)
            def _():
                out_wait(b, rt0 + k)

            @pl.when(k + NB1 < nblk)
            def _():
                in_start(b, rt0 + k + NB1)


def _g_body(xt_hbm, st_hbm, out_hbm, idx_v, gbuf, obuf, gsem, osem):
    c = lax.axis_index("c")
    s = lax.axis_index("s")
    wid = s * NC + c

    iot = lax.iota(jnp.int32, 16)
    ldrow = [16 * h + iot for h in range(2)]
    ricb = [lax.rem(iot + t, 16) for t in range(16)]
    dtv = [(16 * h + iot) // 8 for h in range(2)]
    div = lax.rem(iot, 8)

    pltpu.sync_copy(xt_hbm.at[:, pl.ds(wid * 512, 512)], idx_v)

    def fbt(j):
        return j // 4, 4 * wid + lax.rem(j, 4), lax.rem(j, 4)

    def g_start(b, j):
        f, bt, bta = fbt(j)
        pltpu.make_async_copy(
            st_hbm.at[idx_v.at[f, pl.ds(bta * 128, 128)]], gbuf.at[b],
            gsem.at[b]).start()

    def g_wait(b, j):
        f, bt, bta = fbt(j)
        pltpu.make_async_copy(
            st_hbm.at[idx_v.at[f, pl.ds(bta * 128, 128)]], gbuf.at[b],
            gsem.at[b]).wait()

    def o_start(b, j):
        f, bt, bta = fbt(j)
        pltpu.make_async_copy(
            obuf.at[b], out_hbm.at[f, :, bt], osem.at[b]).start()

    def o_wait(b, j):
        f, bt, bta = fbt(j)
        pltpu.make_async_copy(
            obuf.at[b], out_hbm.at[f, :, bt], osem.at[b]).wait()

    def transpose(b):
        @plsc.parallel_loop(0, 8, unroll=2)
        def _g(g):
            biv = [ricb[t] + 16 * g for t in range(16)]
            for h in range(2):
                vals = [plsc.load_gather(gbuf.at[b], [biv[t], ldrow[h]])
                        for t in range(16)]
                for t in range(16):
                    plsc.store_scatter(obuf.at[b], [dtv[h], div, biv[t]],
                                       vals[t])

    for b in range(NBUF):
        g_start(b, b)

    @pl.loop(0, NGRP * NBUF, step=NBUF)
    def _grp(g):
        for b in range(NBUF):
            g_wait(b, g + b)
            transpose(b)
            o_start(b, g + b)
        for b in range(NBUF):
            o_wait(b, g + b)

            @pl.when(g + NBUF + b < CPW)
            def _():
                g_start(b, g + NBUF + b)


def _mesh():
    return plsc.VectorSubcoreMesh(
        core_axis_name="c", subcore_axis_name="s",
        num_cores=NC, num_subcores=NS)


@jax.jit
def _lookup(x_flat, tt):
    k1 = functools.partial(
        pl.kernel,
        out_type=jax.ShapeDtypeStruct((250016, 128), jnp.float32),
        mesh=_mesh(),
        compiler_params=pltpu.CompilerParams(needs_layout_passes=False),
        scratch_types=[
            pltpu.VMEM((NB1, 32, 128), jnp.float32),
            pltpu.VMEM((NB1, 32, 128), jnp.float32),
            pltpu.SemaphoreType.DMA((NB1,)),
            pltpu.SemaphoreType.DMA((NB1,)),
        ],
    )(_t_body)
    staging = k1(tt).reshape(1000064, 32)
    k2 = functools.partial(
        pl.kernel,
        out_type=jax.ShapeDtypeStruct((FIELDS, 4, 128, 8, 128), jnp.float32),
        mesh=_mesh(),
        compiler_params=pltpu.CompilerParams(use_tc_tiling_on_sc=False, needs_layout_passes=False),
        scratch_types=[
            pltpu.VMEM((FIELDS, 512), jnp.int32),
            pltpu.VMEM((NBUF, CHUNK, DIM), jnp.float32),
            pltpu.VMEM((NBUF, 4, 8, 128), jnp.float32),
            pltpu.SemaphoreType.DMA((NBUF,)),
            pltpu.SemaphoreType.DMA((NBUF,)),
        ],
    )(_g_body)
    return k2(x_flat, staging)


def kernel(x, table):
    xt = x.T.astype(jnp.int32)
    out5 = _lookup(xt, table.T)
    return jnp.transpose(out5, (2, 4, 0, 1, 3)).reshape(BATCH, FIELDS, DIM)


# speedup vs baseline: 2.4586x; 1.5234x over previous
import functools

import jax
import jax.numpy as jnp
from jax import lax
from jax.experimental import pallas as pl
from jax.experimental.pallas import tpu as pltpu
from jax.experimental.pallas import tpu_sc as plsc

NC = 2
NS = 16
NW = NC * NS

BATCH = 16384
FIELDS = 26
DIM = 32
B = BATCH * FIELDS
BPW = B // NW
CHUNK = 128
CPW = BPW // CHUNK
NBUF = 4
NGRP = CPW // NBUF

NB1 = 4

assert CPW * CHUNK == BPW and NGRP * NBUF == CPW


def _t_body(tt_hbm, st_hbm, inb, outb, gsem, osem):
    c = lax.axis_index("c")
    s = lax.axis_index("s")
    wid = s * NC + c
    nblk = jnp.where(wid < 5, 245, 244)
    rt0 = 244 * wid + jnp.minimum(wid, 5)

    iot = lax.iota(jnp.int32, 16)
    ldrow = [16 * h + iot for h in range(2)]
    ricb = [lax.rem(iot + t, 16) for t in range(16)]
    strow = [r // 4 for r in ricb]
    stcol = [[lax.rem(ricb[t], 4) * 32 + 16 * h + iot
              for t in range(16)] for h in range(2)]

    def in_start(b, rt):
        pltpu.make_async_copy(
            tt_hbm.at[:, pl.ds(rt * 128, 128)], inb.at[b], gsem.at[b]).start()

    def in_wait(b, rt):
        pltpu.make_async_copy(
            tt_hbm.at[:, pl.ds(rt * 128, 128)], inb.at[b], gsem.at[b]).wait()

    def out_start(b, rt):
        pltpu.make_async_copy(
            outb.at[b], st_hbm.at[pl.ds(rt * 32, 32)], osem.at[b]).start()

    def out_wait(b, rt):
        pltpu.make_async_copy(
            outb.at[b], st_hbm.at[pl.ds(rt * 32, 32)], osem.at[b]).wait()

    def transpose(b):
        @plsc.parallel_loop(0, 8, unroll=4)
        def _g(g):
            ric = [ricb[t] + 16 * g for t in range(16)]
            srw = [strow[t] + 4 * g for t in range(16)]
            for h in range(2):
                vals = [plsc.load_gather(inb.at[b], [ldrow[h], ric[t]])
                        for t in range(16)]
                for t in range(16):
                    plsc.store_scatter(outb.at[b], [srw[t], stcol[h][t]], vals[t])

    for b in range(NB1):
        @pl.when(b < nblk)
        def _():
            in_start(b, rt0 + b)

    @pl.loop(0, 248, step=NB1)
    def _grp(g):
        for b in range(NB1):
            k = g + b

            @pl.when(k < nblk)
            def _():
                in_wait(b, rt0 + k)
                transpose(b)
                out_start(b, rt0 + k)
        for b in range(NB1):
            k = g + b

            @pl.when(k < nblk)
            def _():
                out_wait(b, rt0 + k)

            @pl.when(k + NB1 < nblk)
            def _():
                in_start(b, rt0 + k + NB1)


def _g_body(xt_hbm, st_hbm, out_hbm, idx_v, gbuf, obuf, gsem, osem):
    c = lax.axis_index("c")
    s = lax.axis_index("s")
    wid = s * NC + c

    iot = lax.iota(jnp.int32, 16)
    ldrow = [16 * h + iot for h in range(2)]
    ricb = [lax.rem(iot + t, 16) for t in range(16)]
    dtv = [(16 * h + iot) // 8 for h in range(2)]
    div = lax.rem(iot, 8)

    pltpu.sync_copy(xt_hbm.at[:, pl.ds(wid * 512, 512)], idx_v)

    def fbt(j):
        return j // 4, 4 * wid + lax.rem(j, 4), lax.rem(j, 4)

    def g_start(b, j):
        f, bt, bta = fbt(j)
        pltpu.make_async_copy(
            st_hbm.at[idx_v.at[f, pl.ds(bta * 128, 128)]], gbuf.at[b],
            gsem.at[b]).start()

    def g_wait(b, j):
        f, bt, bta = fbt(j)
        pltpu.make_async_copy(
            st_hbm.at[idx_v.at[f, pl.ds(bta * 128, 128)]], gbuf.at[b],
            gsem.at[b]).wait()

    def o_start(b, j):
        f, bt, bta = fbt(j)
        pltpu.make_async_copy(
            obuf.at[b], out_hbm.at[f, :, bt], osem.at[b]).start()

    def o_wait(b, j):
        f, bt, bta = fbt(j)
        pltpu.make_async_copy(
            obuf.at[b], out_hbm.at[f, :, bt], osem.at[b]).wait()

    def transpose(b):
        @plsc.parallel_loop(0, 8, unroll=2)
        def _g(g):
            biv = [ricb[t] + 16 * g for t in range(16)]
            for h in range(2):
                vals = [plsc.load_gather(gbuf.at[b], [biv[t], ldrow[h]])
                        for t in range(16)]
                for t in range(16):
                    plsc.store_scatter(obuf.at[b], [dtv[h], div, biv[t]],
                                       vals[t])

    for b in range(NBUF):
        g_start(b, b)

    @pl.loop(0, NGRP * NBUF, step=NBUF)
    def _grp(g):
        for b in range(NBUF):
            g_wait(b, g + b)
            transpose(b)
            o_start(b, g + b)
        for b in range(NBUF):
            o_wait(b, g + b)

            @pl.when(g + NBUF + b < CPW)
            def _():
                g_start(b, g + NBUF + b)


def _mesh():
    return plsc.VectorSubcoreMesh(
        core_axis_name="c", subcore_axis_name="s",
        num_cores=NC, num_subcores=NS)


@jax.jit
def _lookup(x_flat, tt):
    k1 = functools.partial(
        pl.kernel,
        out_type=jax.ShapeDtypeStruct((250016, 128), jnp.float32),
        mesh=_mesh(),
        compiler_params=pltpu.CompilerParams(needs_layout_passes=False),
        scratch_types=[
            pltpu.VMEM((NB1, 32, 128), jnp.float32),
            pltpu.VMEM((NB1, 32, 128), jnp.float32),
            pltpu.SemaphoreType.DMA((NB1,)),
            pltpu.SemaphoreType.DMA((NB1,)),
        ],
    )(_t_body)
    staging = k1(tt).reshape(1000064, 32)
    k2 = functools.partial(
        pl.kernel,
        out_type=jax.ShapeDtypeStruct((FIELDS, 4, 128, 8, 128), jnp.float32),
        mesh=_mesh(),
        compiler_params=pltpu.CompilerParams(use_tc_tiling_on_sc=False, needs_layout_passes=False),
        scratch_types=[
            pltpu.VMEM((FIELDS, 512), jnp.int32),
            pltpu.VMEM((NBUF, CHUNK, DIM), jnp.float32),
            pltpu.VMEM((NBUF, 4, 8, 128), jnp.float32),
            pltpu.SemaphoreType.DMA((NBUF,)),
            pltpu.SemaphoreType.DMA((NBUF,)),
        ],
    )(_g_body)
    return k2(x_flat, staging)


def kernel(x, table):
    xt = x.T.astype(jnp.int32)
    out5 = _lookup(xt, table.T)
    return jnp.transpose(out5, (2, 4, 0, 1, 3)).reshape(BATCH, FIELDS, DIM)


# speedup vs baseline: 3.2966x; 1.3408x over previous
import functools

import jax
import jax.numpy as jnp
from jax import lax
from jax.experimental import pallas as pl
from jax.experimental.pallas import tpu as pltpu
from jax.experimental.pallas import tpu_sc as plsc

NC = 2
NS = 16
NW = NC * NS

BATCH = 16384
FIELDS = 26
DIM = 32
B = BATCH * FIELDS
BPW = B // NW
CHUNK = 128
CPW = BPW // CHUNK
NBUF = 4
NGRP = CPW // NBUF

NB1 = 4

assert CPW * CHUNK == BPW and NGRP * NBUF == CPW


def _t_body(tt_hbm, st_hbm, inb, outb, gsem, osem):
    c = lax.axis_index("c")
    s = lax.axis_index("s")
    wid = s * NC + c
    nblk = jnp.where(wid < 5, 245, 244)
    rt0 = 244 * wid + jnp.minimum(wid, 5)

    iot = lax.iota(jnp.int32, 16)
    ldrow = [16 * h + iot for h in range(2)]
    ricb = [lax.rem(iot + t, 16) for t in range(16)]
    strow = [r // 4 for r in ricb]
    stcol = [[lax.rem(ricb[t], 4) * 32 + 16 * h + iot
              for t in range(16)] for h in range(2)]

    def in_start(b, rt):
        pltpu.make_async_copy(
            tt_hbm.at[:, pl.ds(rt * 128, 128)], inb.at[b], gsem.at[b]).start()

    def in_wait(b, rt):
        pltpu.make_async_copy(
            tt_hbm.at[:, pl.ds(rt * 128, 128)], inb.at[b], gsem.at[b]).wait()

    def out_start(b, rt):
        pltpu.make_async_copy(
            outb.at[b], st_hbm.at[pl.ds(rt * 32, 32)], osem.at[b]).start()

    def out_wait(b, rt):
        pltpu.make_async_copy(
            outb.at[b], st_hbm.at[pl.ds(rt * 32, 32)], osem.at[b]).wait()

    def transpose(b):
        @plsc.parallel_loop(0, 8, unroll=4)
        def _g(g):
            ric = [ricb[t] + 16 * g for t in range(16)]
            srw = [strow[t] + 4 * g for t in range(16)]
            for h in range(2):
                vals = [plsc.load_gather(inb.at[b], [ldrow[h], ric[t]])
                        for t in range(16)]
                for t in range(16):
                    plsc.store_scatter(outb.at[b], [srw[t], stcol[h][t]], vals[t])

    for b in range(NB1):
        @pl.when(b < nblk)
        def _():
            in_start(b, rt0 + b)

    @pl.loop(0, 248, step=NB1)
    def _grp(g):
        for b in range(NB1):
            k = g + b

            @pl.when(k < nblk)
            def _():
                in_wait(b, rt0 + k)
                transpose(b)
                out_start(b, rt0 + k)
        for b in range(NB1):
            k = g + b

            @pl.when(k < nblk)
            def _():
                out_wait(b, rt0 + k)

            @pl.when(k + NB1 < nblk)
            def _():
                in_start(b, rt0 + k + NB1)


def _g_body(xt_hbm, st_hbm, out_hbm, idx_v, gbuf, obuf, gsem, osem):
    c = lax.axis_index("c")
    s = lax.axis_index("s")
    wid = s * NC + c

    iot = lax.iota(jnp.int32, 16)
    ldrow = [16 * h + iot for h in range(2)]
    ricb = [lax.rem(iot + t, 16) for t in range(16)]
    dtv = [(16 * h + iot) // 8 for h in range(2)]
    div = lax.rem(iot, 8)

    pltpu.sync_copy(xt_hbm.at[:, pl.ds(wid * 512, 512)], idx_v)

    def fbt(j):
        return j // 4, 4 * wid + lax.rem(j, 4), lax.rem(j, 4)

    def g_start(b, j):
        f, bt, bta = fbt(j)
        pltpu.make_async_copy(
            st_hbm.at[idx_v.at[f, pl.ds(bta * 128, 128)]], gbuf.at[b],
            gsem.at[b]).start()

    def g_wait(b, j):
        f, bt, bta = fbt(j)
        pltpu.make_async_copy(
            st_hbm.at[idx_v.at[f, pl.ds(bta * 128, 128)]], gbuf.at[b],
            gsem.at[b]).wait()

    def o_start(b, j):
        f, bt, bta = fbt(j)
        pltpu.make_async_copy(
            obuf.at[b], out_hbm.at[f, :, bt], osem.at[b]).start()

    def o_wait(b, j):
        f, bt, bta = fbt(j)
        pltpu.make_async_copy(
            obuf.at[b], out_hbm.at[f, :, bt], osem.at[b]).wait()

    def transpose(b):
        @plsc.parallel_loop(0, 8, unroll=4)
        def _g(g):
            biv = [ricb[t] + 16 * g for t in range(16)]
            for h in range(2):
                vals = [plsc.load_gather(gbuf.at[b], [biv[t], ldrow[h]])
                        for t in range(16)]
                for t in range(16):
                    plsc.store_scatter(obuf.at[b], [dtv[h], div, biv[t]],
                                       vals[t])

    for b in range(NBUF):
        g_start(b, b)

    @pl.loop(0, NGRP * NBUF, step=NBUF)
    def _grp(g):
        for b in range(NBUF):
            g_wait(b, g + b)
            transpose(b)
            o_start(b, g + b)
        for b in range(NBUF):
            o_wait(b, g + b)

            @pl.when(g + NBUF + b < CPW)
            def _():
                g_start(b, g + NBUF + b)


def _mesh():
    return plsc.VectorSubcoreMesh(
        core_axis_name="c", subcore_axis_name="s",
        num_cores=NC, num_subcores=NS)


@jax.jit
def _lookup(x_flat, tt):
    k1 = functools.partial(
        pl.kernel,
        out_type=jax.ShapeDtypeStruct((250016, 128), jnp.float32),
        mesh=_mesh(),
        compiler_params=pltpu.CompilerParams(needs_layout_passes=False),
        scratch_types=[
            pltpu.VMEM((NB1, 32, 128), jnp.float32),
            pltpu.VMEM((NB1, 32, 128), jnp.float32),
            pltpu.SemaphoreType.DMA((NB1,)),
            pltpu.SemaphoreType.DMA((NB1,)),
        ],
    )(_t_body)
    staging = k1(tt).reshape(1000064, 32)
    k2 = functools.partial(
        pl.kernel,
        out_type=jax.ShapeDtypeStruct((FIELDS, 4, 128, 8, 128), jnp.float32),
        mesh=_mesh(),
        compiler_params=pltpu.CompilerParams(use_tc_tiling_on_sc=False, needs_layout_passes=False),
        scratch_types=[
            pltpu.VMEM((FIELDS, 512), jnp.int32),
            pltpu.VMEM((NBUF, CHUNK, DIM), jnp.float32),
            pltpu.VMEM((NBUF, 4, 8, 128), jnp.float32),
            pltpu.SemaphoreType.DMA((NBUF,)),
            pltpu.SemaphoreType.DMA((NBUF,)),
        ],
    )(_g_body)
    return k2(x_flat, staging)


def kernel(x, table):
    xt = x.T.astype(jnp.int32)
    out5 = _lookup(xt, table.T)
    return jnp.transpose(out5, (2, 4, 0, 1, 3)).reshape(BATCH, FIELDS, DIM)
